# baseline (speedup 1.0000x reference)
import functools

import jax
import jax.numpy as jnp
from jax import lax
from jax.experimental import pallas as pl
from jax.experimental.pallas import tpu as pltpu
from jax.experimental.pallas import tpu_sc as plsc

_VOCAB_PAD = 32
_CHUNK = 128


def _fuse_body(table_ref, w_ref, b_ref, out_ref):
    emb = table_ref[...]
    w = w_ref[...]
    b = b_ref[...]
    out = jnp.dot(emb, w.T, preferred_element_type=jnp.float32) + b
    nrm = jnp.sqrt(jnp.sum(out * out, axis=1, keepdims=True))
    out_ref[...] = out / jnp.maximum(nrm, 1e-12)


def _fused_table(table, W, b):
    vocab, d = table.shape
    table_p = jnp.zeros((_VOCAB_PAD, d), jnp.float32).at[:vocab].set(table)
    return pl.pallas_call(
        _fuse_body,
        out_shape=jax.ShapeDtypeStruct((_VOCAB_PAD, d), jnp.float32),
    )(table_p, W, b.reshape(1, d))


def kernel(indices, table, W, b):
    batch = indices.shape[0]
    d = table.shape[1]
    info = plsc.get_sparse_core_info()
    nc, ns = info.num_cores, info.num_subcores
    nw = nc * ns
    bpw = batch // nw
    nchunks = bpw // _CHUNK

    fused = _fused_table(table, W, b)
    idx3 = indices.reshape(nw, nchunks, _CHUNK)

    mesh = plsc.VectorSubcoreMesh(core_axis_name="c", subcore_axis_name="s")

    @functools.partial(
        pl.kernel,
        mesh=mesh,
        compiler_params=pltpu.CompilerParams(use_tc_tiling_on_sc=False),
        out_type=jax.ShapeDtypeStruct((batch, d), jnp.float32),
        scratch_types=[
            pltpu.VMEM((nchunks, _CHUNK), jnp.int32),
            pltpu.VMEM((bpw, d), jnp.float32),
            pltpu.SemaphoreType.DMA,
        ],
    )
    def _gather(fused_hbm, idx_hbm, out_hbm, idx_v, rows_v, sem):
        wid = lax.axis_index("s") * nc + lax.axis_index("c")
        pltpu.sync_copy(idx_hbm.at[wid], idx_v)
        copies = [
            pltpu.async_copy(
                fused_hbm.at[idx_v.at[j]],
                rows_v.at[pl.ds(j * _CHUNK, _CHUNK)],
                sem,
            )
            for j in range(nchunks)
        ]
        for c in copies:
            c.wait()
        pltpu.sync_copy(rows_v, out_hbm.at[pl.ds(wid * bpw, bpw)])

    return _gather(fused, idx3)


# speedup vs baseline: 1.6084x; 1.6084x over previous
import functools

import jax
import jax.numpy as jnp
from jax import lax
from jax.experimental import pallas as pl
from jax.experimental.pallas import tpu as pltpu
from jax.experimental.pallas import tpu_sc as plsc

_VOCAB_PAD = 32
_LANES = 16


def _fuse_body(table_ref, w_ref, b_ref, out_ref):
    emb = table_ref[...]
    w = w_ref[...]
    b = b_ref[...]
    out = jnp.dot(emb, w.T, preferred_element_type=jnp.float32) + b
    nrm = jnp.sqrt(jnp.sum(out * out, axis=1, keepdims=True))
    out_ref[...] = out / jnp.maximum(nrm, 1e-12)


def _fused_table(table, W, b):
    vocab, d = table.shape
    table_p = jnp.zeros((_VOCAB_PAD, d), jnp.float32).at[:vocab].set(table)
    return pl.pallas_call(
        _fuse_body,
        out_shape=jax.ShapeDtypeStruct((_VOCAB_PAD, d), jnp.float32),
    )(table_p, W, b.reshape(1, d))


def kernel(indices, table, W, b):
    batch = indices.shape[0]
    d = table.shape[1]
    info = plsc.get_sparse_core_info()
    nc, ns = info.num_cores, info.num_subcores
    nw = nc * ns
    bpw = batch // nw
    ngroups = bpw // _LANES
    tab_elems = _VOCAB_PAD * d

    fused = _fused_table(table, W, b).reshape(tab_elems)
    idx2 = indices.reshape(nw, bpw)

    mesh = plsc.VectorSubcoreMesh(core_axis_name="c", subcore_axis_name="s")

    @functools.partial(
        pl.kernel,
        mesh=mesh,
        compiler_params=pltpu.CompilerParams(
            use_tc_tiling_on_sc=False, needs_layout_passes=False
        ),
        out_type=jax.ShapeDtypeStruct((batch * d,), jnp.float32),
        scratch_types=[
            pltpu.VMEM((tab_elems,), jnp.float32),
            pltpu.VMEM((bpw,), jnp.int32),
            pltpu.VMEM((bpw * d,), jnp.float32),
        ],
    )
    def _gather(fused_hbm, idx_hbm, out_hbm, tab_v, idx_v, rows_v):
        wid = lax.axis_index("s") * nc + lax.axis_index("c")
        pltpu.sync_copy(fused_hbm, tab_v)
        pltpu.sync_copy(idx_hbm.at[wid], idx_v)
        lane = lax.iota(jnp.int32, _LANES)
        lane_row = lane * d

        def group_body(g, carry):
            ridx = idx_v[pl.ds(pl.multiple_of(g * _LANES, _LANES), _LANES)]
            src_base = ridx * d
            dst_base = g * (_LANES * d) + lane_row
            for dcol in range(d):
                col = plsc.load_gather(tab_v, [src_base + dcol])
                plsc.store_scatter(rows_v, [dst_base + dcol], col)
            return carry

        lax.fori_loop(0, ngroups, group_body, 0, unroll=8)
        pltpu.sync_copy(rows_v, out_hbm.at[pl.ds(wid * bpw * d, bpw * d)])

    return _gather(fused, idx2).reshape(batch, d)
